# baseline (speedup 1.0000x reference)
import functools

import jax
import jax.numpy as jnp
from jax import lax
from jax.experimental import pallas as pl
from jax.experimental.pallas import tpu as pltpu
from jax.experimental.pallas import tpu_sc as plsc

_ROWS_PER_BLOCK = 8


def kernel(x, pos_emb):
    batch, seq_len, d = x.shape
    info = plsc.get_sparse_core_info()
    lanes = info.num_lanes
    nw = info.num_cores * info.num_subcores
    chunk = (seq_len // nw) * d
    blk = _ROWS_PER_BLOCK * d
    nblk = chunk // blk

    xf = x.reshape(batch, seq_len * d)
    pf = pos_emb.reshape(-1)

    mesh = plsc.VectorSubcoreMesh(core_axis_name="c", subcore_axis_name="s")

    @functools.partial(
        pl.kernel,
        out_type=jax.ShapeDtypeStruct((batch, seq_len * d), jnp.float32),
        mesh=mesh,
        scratch_types=[
            pltpu.VMEM((blk,), jnp.float32),
            pltpu.VMEM((batch, blk), jnp.float32),
        ],
    )
    def sc_kernel(x_hbm, pos_hbm, out_hbm, ebuf, xbuf):
        wid = lax.axis_index("s") * info.num_cores + lax.axis_index("c")
        base = wid * chunk

        def block_body(j, carry):
            off = base + j * blk
            pltpu.sync_copy(pos_hbm.at[pl.ds(off, blk)], ebuf)
            for b in range(batch):
                pltpu.sync_copy(x_hbm.at[b, pl.ds(off, blk)], xbuf.at[b])

            @plsc.parallel_loop(0, blk, step=lanes, unroll=4)
            def add_body(i):
                ev = ebuf[pl.ds(i, lanes)]
                for b in range(batch):
                    xbuf[b, pl.ds(i, lanes)] = xbuf[b, pl.ds(i, lanes)] + ev

            for b in range(batch):
                pltpu.sync_copy(xbuf.at[b], out_hbm.at[b, pl.ds(off, blk)])
            return carry

        lax.fori_loop(0, nblk, block_body, 0)

    return sc_kernel(xf, pf).reshape(x.shape)


# speedup vs baseline: 1.3480x; 1.3480x over previous
import jax
import jax.numpy as jnp
from jax import lax
from jax.experimental import pallas as pl
from jax.experimental.pallas import tpu as pltpu
from jax.experimental.pallas import tpu_sc as plsc

_ROWS_PER_BLOCK = 8
_NSETS = 3


def kernel(x, pos_emb):
    batch, seq_len, d = x.shape
    info = plsc.get_sparse_core_info()
    lanes = info.num_lanes
    nw = info.num_cores * info.num_subcores
    chunk = (seq_len // nw) * d
    blk = _ROWS_PER_BLOCK * d
    nblk = chunk // blk

    xf = x.reshape(batch, seq_len * d)
    pf = pos_emb.reshape(-1)

    mesh = plsc.VectorSubcoreMesh(core_axis_name="c", subcore_axis_name="s")

    @pl.kernel(
        out_type=jax.ShapeDtypeStruct((batch, seq_len * d), jnp.float32),
        mesh=mesh,
        scratch_types=[
            pltpu.VMEM((_NSETS * blk,), jnp.float32),
            pltpu.VMEM((_NSETS * batch * blk,), jnp.float32),
        ]
        + [pltpu.SemaphoreType.DMA] * (2 * _NSETS),
    )
    def sc_kernel(x_hbm, pos_hbm, out_hbm, ebuf, xbuf, *sems):
        in_sems, out_sems = sems[:_NSETS], sems[_NSETS:]
        wid = lax.axis_index("s") * info.num_cores + lax.axis_index("c")
        base = wid * chunk

        def issue_in(j):
            s = j % _NSETS
            off = base + j * blk
            descs = [
                pltpu.async_copy(
                    pos_hbm.at[pl.ds(off, blk)],
                    ebuf.at[pl.ds(s * blk, blk)],
                    in_sems[s],
                )
            ]
            for b in range(batch):
                descs.append(
                    pltpu.async_copy(
                        x_hbm.at[b, pl.ds(off, blk)],
                        xbuf.at[pl.ds((s * batch + b) * blk, blk)],
                        in_sems[s],
                    )
                )
            return descs

        def issue_out(j):
            s = j % _NSETS
            off = base + j * blk
            return [
                pltpu.async_copy(
                    xbuf.at[pl.ds((s * batch + b) * blk, blk)],
                    out_hbm.at[b, pl.ds(off, blk)],
                    out_sems[s],
                )
                for b in range(batch)
            ]

        pend_in = {0: issue_in(0)}
        if nblk > 1:
            pend_in[1] = issue_in(1)
        pend_out = {}

        for j in range(nblk):
            s = j % _NSETS
            for dsc in pend_in.pop(j):
                dsc.wait()

            @plsc.parallel_loop(0, blk, step=lanes, unroll=4)
            def add_body(i, s=s):
                ev = ebuf[pl.ds(s * blk + i, lanes)]
                for b in range(batch):
                    o = (s * batch + b) * blk
                    xbuf[pl.ds(o + i, lanes)] = xbuf[pl.ds(o + i, lanes)] + ev

            pend_out[j] = issue_out(j)
            if j + 2 < nblk:
                if j - 1 in pend_out:
                    for dsc in pend_out.pop(j - 1):
                        dsc.wait()
                pend_in[j + 2] = issue_in(j + 2)

        for jj in sorted(pend_out):
            for dsc in pend_out[jj]:
                dsc.wait()

    return sc_kernel(xf, pf).reshape(x.shape)


# speedup vs baseline: 3.5856x; 2.6600x over previous
import jax
import jax.numpy as jnp
from jax import lax
from jax.experimental import pallas as pl
from jax.experimental.pallas import tpu as pltpu
from jax.experimental.pallas import tpu_sc as plsc

_R = 8
_NSETS = 3


def kernel(x, pos_emb):
    batch, seq_len, d = x.shape
    info = plsc.get_sparse_core_info()
    lanes = info.num_lanes
    nw = info.num_cores * info.num_subcores
    rows_per_worker = seq_len // nw
    nblk = rows_per_worker // _R
    vecs_per_row = d // lanes

    mesh = plsc.VectorSubcoreMesh(core_axis_name="c", subcore_axis_name="s")

    @pl.kernel(
        out_type=jax.ShapeDtypeStruct(x.shape, jnp.float32),
        mesh=mesh,
        scratch_types=[
            pltpu.VMEM((_NSETS * _R, d), jnp.float32),
            pltpu.VMEM((_NSETS * batch * _R, d), jnp.float32),
        ]
        + [pltpu.SemaphoreType.DMA] * (2 * _NSETS),
    )
    def sc_kernel(x_hbm, pos_hbm, out_hbm, ebuf, xbuf, *sems):
        in_sems, out_sems = sems[:_NSETS], sems[_NSETS:]
        wid = lax.axis_index("s") * info.num_cores + lax.axis_index("c")
        row0 = wid * rows_per_worker

        def issue_in(j):
            s = j % _NSETS
            row = row0 + j * _R
            descs = [
                pltpu.async_copy(
                    pos_hbm.at[pl.ds(row, _R)],
                    ebuf.at[pl.ds(s * _R, _R)],
                    in_sems[s],
                )
            ]
            for b in range(batch):
                descs.append(
                    pltpu.async_copy(
                        x_hbm.at[b, pl.ds(row, _R)],
                        xbuf.at[pl.ds((s * batch + b) * _R, _R)],
                        in_sems[s],
                    )
                )
            return descs

        def issue_out(j):
            s = j % _NSETS
            row = row0 + j * _R
            return [
                pltpu.async_copy(
                    xbuf.at[pl.ds((s * batch + b) * _R, _R)],
                    out_hbm.at[b, pl.ds(row, _R)],
                    out_sems[s],
                )
                for b in range(batch)
            ]

        pend_in = {0: issue_in(0)}
        if nblk > 1:
            pend_in[1] = issue_in(1)
        pend_out = {}

        for j in range(nblk):
            s = j % _NSETS
            for dsc in pend_in.pop(j):
                dsc.wait()

            @plsc.parallel_loop(0, _R * vecs_per_row, step=1, unroll=4)
            def add_body(i, s=s):
                r = i // vecs_per_row
                c = (i % vecs_per_row) * lanes
                ev = ebuf[s * _R + r, pl.ds(c, lanes)]
                for b in range(batch):
                    rr = (s * batch + b) * _R + r
                    xbuf[rr, pl.ds(c, lanes)] = xbuf[rr, pl.ds(c, lanes)] + ev

            pend_out[j] = issue_out(j)
            if j + 2 < nblk:
                if j - 1 in pend_out:
                    for dsc in pend_out.pop(j - 1):
                        dsc.wait()
                pend_in[j + 2] = issue_in(j + 2)

        for jj in sorted(pend_out):
            for dsc in pend_out[jj]:
                dsc.wait()

    return sc_kernel(x, pos_emb)


# speedup vs baseline: 3.6220x; 1.0102x over previous
import jax
import jax.numpy as jnp
from jax import lax
from jax.experimental import pallas as pl
from jax.experimental.pallas import tpu as pltpu
from jax.experimental.pallas import tpu_sc as plsc

_R = 8
_NSETS = 3


def kernel(x, pos_emb):
    batch, seq_len, d = x.shape
    info = plsc.get_sparse_core_info()
    lanes = info.num_lanes
    nw = info.num_cores * info.num_subcores
    rows_per_worker = seq_len // nw
    nblk = rows_per_worker // _R
    vecs_per_row = d // lanes

    mesh = plsc.VectorSubcoreMesh(core_axis_name="c", subcore_axis_name="s")

    @pl.kernel(
        out_type=jax.ShapeDtypeStruct(x.shape, jnp.float32),
        mesh=mesh,
        scratch_types=[
            pltpu.VMEM((_NSETS * _R, d), jnp.float32),
            pltpu.VMEM((_NSETS * batch * _R, d), jnp.float32),
        ]
        + [pltpu.SemaphoreType.DMA] * (2 * _NSETS),
    )
    def sc_kernel(x_hbm, pos_hbm, out_hbm, ebuf, xbuf, *sems):
        in_sems, out_sems = sems[:_NSETS], sems[_NSETS:]
        wid = lax.axis_index("s") * info.num_cores + lax.axis_index("c")
        row0 = wid * rows_per_worker

        def issue_in(j):
            s = j % _NSETS
            row = row0 + j * _R
            descs = [
                pltpu.async_copy(
                    pos_hbm.at[pl.ds(row, _R)],
                    ebuf.at[pl.ds(s * _R, _R)],
                    in_sems[s],
                )
            ]
            for b in range(batch):
                descs.append(
                    pltpu.async_copy(
                        x_hbm.at[b, pl.ds(row, _R)],
                        xbuf.at[pl.ds((s * batch + b) * _R, _R)],
                        in_sems[s],
                    )
                )
            return descs

        def issue_out(j):
            s = j % _NSETS
            row = row0 + j * _R
            return [
                pltpu.async_copy(
                    xbuf.at[pl.ds((s * batch + b) * _R, _R)],
                    out_hbm.at[b, pl.ds(row, _R)],
                    out_sems[s],
                )
                for b in range(batch)
            ]

        pend_in = {0: issue_in(0)}
        if nblk > 1:
            pend_in[1] = issue_in(1)
        pend_out = {}

        for j in range(nblk):
            s = j % _NSETS
            for dsc in pend_in.pop(j):
                dsc.wait()

            @plsc.parallel_loop(0, _R * vecs_per_row, step=1, unroll=4)
            def add_body(i, s=s):
                r = i // vecs_per_row
                c = (i % vecs_per_row) * lanes
                ev = ebuf[s * _R + r, pl.ds(c, lanes)]
                for b in range(batch):
                    rr = (s * batch + b) * _R + r
                    plsc.addupdate(xbuf.at[rr, pl.ds(c, lanes)], ev)

            pend_out[j] = issue_out(j)
            if j + 2 < nblk:
                if j - 1 in pend_out:
                    for dsc in pend_out.pop(j - 1):
                        dsc.wait()
                pend_in[j + 2] = issue_in(j + 2)

        for jj in sorted(pend_out):
            for dsc in pend_out[jj]:
                dsc.wait()

    return sc_kernel(x, pos_emb)
